# baseline (speedup 1.0000x reference)
import functools

import jax
import jax.numpy as jnp
from jax import lax
from jax.experimental import pallas as pl
from jax.experimental.pallas import tpu as pltpu
from jax.experimental.pallas import tpu_sc as plsc

_NSEQ = 64
_VOCAB = 100000
_BUFN = 100096
_NV = _BUFN // 16
_NBINS = 8192
_BIN_SHIFT = 19
_P = 2048
_PV = _P // 16
_KSEL = 1024
_NEG_INF = float("-inf")


def _iota16():
    return lax.iota(jnp.int32, 16)


def _full_f(x):
    return jnp.full((16,), x, dtype=jnp.float32)


def _full_i(x):
    return jnp.full((16,), x, dtype=jnp.int32)


def _permute(x, idx):
    dnums = lax.GatherDimensionNumbers(
        offset_dims=(), collapsed_slice_dims=(0,), start_index_map=(0,))
    return lax.gather(x, idx[:, None], dnums, (1,),
                      mode=lax.GatherScatterMode.PROMISE_IN_BOUNDS)


def _splat_lane(x, lane):
    return _permute(x, jnp.full((16,), lane, dtype=jnp.int32))


def _cumsum16(x):
    iota = _iota16()
    zero = jnp.zeros((16,), dtype=x.dtype)
    for d in (1, 2, 4, 8):
        sh = _permute(x, jnp.maximum(iota - d, 0))
        x = x + jnp.where(iota >= d, sh, zero)
    return x


def _read_splat(ref, i):
    base = lax.shift_left(lax.shift_right_logical(i, 4), 4)
    v = ref[pl.ds(base, 16)]
    return _splat_lane(v, lax.bitwise_and(i, 15))


def _sortable(b):
    s = lax.shift_right_arithmetic(b, 31)
    return lax.bitwise_xor(b, lax.bitwise_and(s, jnp.int32(0x7FFFFFFF)))


def _sc_body(logits_hbm, pres_hbm, freq_hbm, rep_hbm, temp_hbm, topp_hbm,
             minp_hbm, ptok_hbm, otok_hbm, topk_hbm, out_hbm,
             buf, hist, cval, cidx, evals, gidx, zsp, sem,
             ptok, otok, praw, oraw,
             pres_v, freq_v, rep_v, temp_v, topp_v, minp_v, topk_v):
    nc = 2
    wid = lax.axis_index("s") * nc + lax.axis_index("c")

    pltpu.sync_copy(pres_hbm, pres_v)
    pltpu.sync_copy(freq_hbm, freq_v)
    pltpu.sync_copy(rep_hbm, rep_v)
    pltpu.sync_copy(temp_hbm, temp_v)
    pltpu.sync_copy(topp_hbm, topp_v)
    pltpu.sync_copy(minp_hbm, minp_v)
    pltpu.sync_copy(topk_hbm, topk_v)

    iota = _iota16()
    zeros_f = _full_f(0.0)
    zeros_i = _full_i(0)
    ones_i = _full_i(1)
    ninf = _full_f(_NEG_INF)

    zlen = _BUFN // 16
    sid = lax.axis_index("s")

    def z_init(i, _):
        buf[pl.ds(i * 16, 16)] = zeros_f
        return 0
    lax.fori_loop(0, zlen // 16, z_init, 0, unroll=8)
    pltpu.sync_copy(buf.at[pl.ds(0, zlen)], zsp.at[pl.ds(sid * zlen, zlen)])
    plsc.subcore_barrier()

    def do_row(rr, _):
        r = wid * 2 + rr

        pltpu.sync_copy(logits_hbm.at[pl.ds(r * _BUFN, _BUFN)], buf)
        pltpu.sync_copy(ptok_hbm.at[pl.ds(r * 512, 512)], ptok)
        pltpu.sync_copy(otok_hbm.at[pl.ds(r * 128, 128)], otok)

        rep16 = _read_splat(rep_v, r)
        freq16 = _read_splat(freq_v, r)
        pres16 = _read_splat(pres_v, r)
        t_raw = _read_splat(temp_v, r)
        t16 = jnp.where(t_raw < _full_f(1e-2), _full_f(1.0), t_raw)
        topp16 = _full_f(1.0) - _read_splat(topp_v, r)
        minp16 = _read_splat(minp_v, r)
        k_clip = jnp.clip(_read_splat(topk_v, r), 1, _P - 1)[0]

        def g_p(i, _):
            praw[pl.ds(i * 16, 16)] = plsc.load_gather(buf, [ptok[pl.ds(i * 16, 16)]])
            return 0
        lax.fori_loop(0, 32, g_p, 0)

        def g_o(i, _):
            oraw[pl.ds(i * 16, 16)] = plsc.load_gather(buf, [otok[pl.ds(i * 16, 16)]])
            return 0
        lax.fori_loop(0, 8, g_o, 0)

        def w_p(i, _):
            x = praw[pl.ds(i * 16, 16)]
            z = jnp.where(x > zeros_f, x / rep16, x * rep16)
            plsc.store_scatter(buf, [ptok[pl.ds(i * 16, 16)]], z)
            return 0
        lax.fori_loop(0, 32, w_p, 0)

        ovs = [otok[pl.ds(j * 16, 16)] for j in range(8)]

        def cnt_body(rot, accs):
            perm = lax.bitwise_and(iota + rot, _full_i(15))
            rbs = [_permute(b, perm) for b in ovs]
            out = []
            for i in range(8):
                a = accs[i]
                for j in range(8):
                    a = a + jnp.where(ovs[i] == rbs[j], ones_i, zeros_i)
                out.append(a)
            return tuple(out)
        counts = lax.fori_loop(0, 16, cnt_body, tuple(zeros_i for _ in range(8)))

        for j in range(8):
            x = oraw[pl.ds(j * 16, 16)]
            z = jnp.where(x > zeros_f, x / rep16, x * rep16)
            z = z - freq16 * counts[j].astype(jnp.float32)
            z = z - pres16
            plsc.store_scatter(buf, [ovs[j]], z)

        def h_zero(i, _):
            hist[pl.ds(i * 16, 16)] = zeros_i
            return 0
        lax.fori_loop(0, _NBINS // 16, h_zero, 0, unroll=8)

        def h_body(i, _):
            x = buf[pl.ds(i * 16, 16)]
            b = lax.shift_right_logical(
                lax.bitcast_convert_type(x, jnp.int32), _BIN_SHIFT)
            plsc.addupdate_scatter(hist, [b], ones_i)
            return 0
        lax.fori_loop(0, _NV, h_body, 0, unroll=8)

        k16 = _full_i(_KSEL)

        def s_pos(v, carry):
            total, ebits = carry
            vi = (_NBINS // 32 - 1) - v
            h = hist[pl.ds(vi * 16, 16)]
            rc = lax.rev(_cumsum16(lax.rev(h, (0,))), (0,))
            tot_here = _splat_lane(rc, 0)
            cv = (total + rc) >= k16
            pc = plsc.all_reduce_population_count(cv)
            b_here = _full_i(vi * 16) + pc - ones_i
            crossing = jnp.logical_and(total < k16, (total + tot_here) >= k16)
            ebits = jnp.where(crossing, lax.shift_left(b_here, _BIN_SHIFT), ebits)
            return total + tot_here, ebits

        def s_neg(v, carry):
            total, ebits = carry
            vi = (_NBINS // 32) + v
            h = hist[pl.ds(vi * 16, 16)]
            pf = _cumsum16(h)
            tot_here = _splat_lane(pf, 15)
            cv = (total + pf) >= k16
            pc = plsc.all_reduce_population_count(cv)
            b_here = _full_i(vi * 16 + 16) - pc
            crossing = jnp.logical_and(total < k16, (total + tot_here) >= k16)
            ebits = jnp.where(
                crossing,
                lax.bitwise_or(lax.shift_left(b_here, _BIN_SHIFT),
                               _full_i((1 << _BIN_SHIFT) - 1)),
                ebits)
            return total + tot_here, ebits

        carry = lax.fori_loop(0, _NBINS // 32, s_pos, (zeros_i, zeros_i), unroll=4)
        _, ebits = lax.fori_loop(0, _NBINS // 32, s_neg, carry, unroll=4)
        thresh16 = lax.bitcast_convert_type(ebits, jnp.float32)

        def c_init(i, _):
            cval[pl.ds(i * 16, 16)] = ninf
            cidx[pl.ds(i * 16, 16)] = _full_i(_VOCAB) + iota
            return 0
        lax.fori_loop(0, _PV + 1, c_init, 0, unroll=4)

        def c_body(i, carry):
            off, iv = carry
            x = buf[pl.ds(i * 16, 16)]
            m = x >= thresh16
            off_use = jnp.minimum(off, _P)
            plsc.store_compressed(cval.at[pl.ds(off_use, 16)], x, mask=m)
            plsc.store_compressed(cidx.at[pl.ds(off_use, 16)], iv, mask=m)
            return off + plsc.all_reduce_population_count(m)[0], iv + _full_i(16)
        lax.fori_loop(0, _NV, c_body, (jnp.int32(0), iota), unroll=4)

        def t_body(i, _):
            cval[pl.ds(i * 16, 16)] = cval[pl.ds(i * 16, 16)] / t16
            return 0
        lax.fori_loop(0, _PV, t_body, 0, unroll=4)

        def vsort_dir(v, kdesc):
            kv = cval[pl.ds(v * 16, 16)]
            vv = cidx[pl.ds(v * 16, 16)]
            sk, sv = plsc.sort_key_val(kv, vv)
            ridx = jnp.where(jnp.full((16,), kdesc, dtype=bool),
                             _full_i(15) - iota, iota)
            cval[pl.ds(v * 16, 16)] = _permute(sk, ridx)
            cidx[pl.ds(v * 16, 16)] = _permute(sv, ridx)

        def init_sort(v, _):
            vsort_dir(v, lax.bitwise_and(v, 1) == 1)
            return 0
        lax.fori_loop(0, _PV, init_sort, 0, unroll=4)

        for k_el in [32, 64, 128, 256, 512, 1024, 2048]:
            kv16 = k_el // 16
            j_el = k_el // 2
            while j_el >= 16:
                jv = j_el // 16
                lg = jv.bit_length() - 1

                def ce_body(p, _, jv=jv, lg=lg, kv16=kv16):
                    v1 = lax.bitwise_or(
                        lax.shift_left(lax.shift_right_logical(p, lg), lg + 1),
                        lax.bitwise_and(p, jv - 1))
                    v2 = v1 + jv
                    asc = lax.bitwise_and(v1, kv16) == 0
                    a = cval[pl.ds(v1 * 16, 16)]
                    b = cval[pl.ds(v2 * 16, 16)]
                    ai = cidx[pl.ds(v1 * 16, 16)]
                    bi = cidx[pl.ds(v2 * 16, 16)]
                    asc16 = jnp.full((16,), asc, dtype=bool)
                    sel = (a <= b) == asc16
                    cval[pl.ds(v1 * 16, 16)] = jnp.where(sel, a, b)
                    cval[pl.ds(v2 * 16, 16)] = jnp.where(sel, b, a)
                    cidx[pl.ds(v1 * 16, 16)] = jnp.where(sel, ai, bi)
                    cidx[pl.ds(v2 * 16, 16)] = jnp.where(sel, bi, ai)
                    return 0
                lax.fori_loop(0, _PV // 2, ce_body, 0, unroll=2)
                j_el //= 2

            def cl_body(v, _, kv16=kv16):
                vsort_dir(v, lax.bitwise_and(v, kv16) != 0)
                return 0
            lax.fori_loop(0, _PV, cl_body, 0, unroll=4)

        pos = _P - k_clip
        base = lax.shift_left(lax.shift_right_logical(pos, 4), 4)
        kth16 = _splat_lane(cval[pl.ds(base, 16)], lax.bitwise_and(pos, 15))
        m16 = _splat_lane(cval[pl.ds(_P - 16, 16)], 15)

        def pa(v, zc):
            x = cval[pl.ds(v * 16, 16)]
            x = jnp.where(x < kth16, ninf, x)
            e = jnp.exp(x - m16)
            evals[pl.ds(v * 16, 16)] = e
            return zc + _splat_lane(_cumsum16(e), 15)
        z1 = lax.fori_loop(0, _PV, pa, zeros_f, unroll=2)

        def pb(v, carry):
            sc, z2c = carry
            e = evals[pl.ds(v * 16, 16)]
            p = e / z1
            cp = _cumsum16(p)
            s = sc + cp
            e2 = jnp.where(s <= topp16, zeros_f, e)
            evals[pl.ds(v * 16, 16)] = e2
            return sc + _splat_lane(cp, 15), z2c + _splat_lane(_cumsum16(e2), 15)
        _, z2 = lax.fori_loop(0, _PV, pb, (zeros_f, zeros_f), unroll=2)

        rhs = minp16 * (_full_f(1.0) / z2)

        def pc(v, z3c):
            e2 = evals[pl.ds(v * 16, 16)]
            p2 = e2 / z2
            e3 = jnp.where(p2 < rhs, zeros_f, e2)
            evals[pl.ds(v * 16, 16)] = e3
            return z3c + _splat_lane(_cumsum16(e3), 15)
        z3 = lax.fori_loop(0, _PV, pc, zeros_f, unroll=2)

        base16 = jnp.full((16,), r * _BUFN, dtype=jnp.int32)

        def pd(v, _):
            evals[pl.ds(v * 16, 16)] = evals[pl.ds(v * 16, 16)] / z3
            gidx[pl.ds(v * 16, 16)] = cidx[pl.ds(v * 16, 16)] + base16
            return 0
        lax.fori_loop(0, _PV, pd, 0, unroll=4)

        pltpu.sync_copy(zsp, out_hbm.at[pl.ds(r * _BUFN, _BUFN)])
        pltpu.async_copy(evals, out_hbm.at[gidx], sem).wait()
        return 0

    lax.fori_loop(0, 2, do_row, 0)


@jax.jit
def kernel(logits, presence_penalties, frequency_penalties,
           repetition_penalties, temperatures, top_p, min_p,
           prompt_tokens, output_tokens, top_k):
    mesh = plsc.VectorSubcoreMesh(core_axis_name="c", subcore_axis_name="s")
    f = pl.kernel(
        _sc_body,
        out_type=jax.ShapeDtypeStruct((_NSEQ * _BUFN,), jnp.float32),
        mesh=mesh,
        compiler_params=pltpu.CompilerParams(needs_layout_passes=False),
        scratch_types=[
            pltpu.VMEM((_BUFN,), jnp.float32),
            pltpu.VMEM((_NBINS,), jnp.int32),
            pltpu.VMEM((_P + 16,), jnp.float32),
            pltpu.VMEM((_P + 16,), jnp.int32),
            pltpu.VMEM((_P,), jnp.float32),
            pltpu.VMEM((_P,), jnp.int32),
            pltpu.VMEM_SHARED((_BUFN,), jnp.float32),
            pltpu.SemaphoreType.DMA,
            pltpu.VMEM((512,), jnp.int32),
            pltpu.VMEM((128,), jnp.int32),
            pltpu.VMEM((512,), jnp.float32),
            pltpu.VMEM((128,), jnp.float32),
            pltpu.VMEM((128,), jnp.float32),
            pltpu.VMEM((128,), jnp.float32),
            pltpu.VMEM((128,), jnp.float32),
            pltpu.VMEM((128,), jnp.float32),
            pltpu.VMEM((128,), jnp.float32),
            pltpu.VMEM((128,), jnp.float32),
            pltpu.VMEM((128,), jnp.int32),
        ],
    )
    logits_p = jnp.pad(logits, ((0, 0), (0, _BUFN - _VOCAB)),
                       constant_values=_NEG_INF).reshape(-1)
    pad1 = lambda a: jnp.pad(a, (0, 128 - _NSEQ))
    out = f(logits_p, pad1(presence_penalties), pad1(frequency_penalties),
            pad1(repetition_penalties), pad1(temperatures), pad1(top_p),
            pad1(min_p), prompt_tokens.reshape(-1), output_tokens.reshape(-1),
            pad1(top_k))
    return out.reshape(_NSEQ, _BUFN)[:, :_VOCAB]


# speedup vs baseline: 1.9895x; 1.9895x over previous
import functools

import jax
import jax.numpy as jnp
from jax import lax
from jax.experimental import pallas as pl
from jax.experimental.pallas import tpu as pltpu
from jax.experimental.pallas import tpu_sc as plsc

_NSEQ = 64
_VOCAB = 100000
_BUFN = 100096
_NV = _BUFN // 16
_NBINS = 8192
_BIN_SHIFT = 19
_P = 2048
_PV = _P // 16
_KSEL = 1024
_NEG_INF = float("-inf")


def _iota16():
    return lax.iota(jnp.int32, 16)


def _full_f(x):
    return jnp.full((16,), x, dtype=jnp.float32)


def _full_i(x):
    return jnp.full((16,), x, dtype=jnp.int32)


def _permute(x, idx):
    dnums = lax.GatherDimensionNumbers(
        offset_dims=(), collapsed_slice_dims=(0,), start_index_map=(0,))
    return lax.gather(x, idx[:, None], dnums, (1,),
                      mode=lax.GatherScatterMode.PROMISE_IN_BOUNDS)


def _splat_lane(x, lane):
    return _permute(x, jnp.full((16,), lane, dtype=jnp.int32))


def _cumsum16(x):
    iota = _iota16()
    zero = jnp.zeros((16,), dtype=x.dtype)
    for d in (1, 2, 4, 8):
        sh = _permute(x, jnp.maximum(iota - d, 0))
        x = x + jnp.where(iota >= d, sh, zero)
    return x


def _read_splat(ref, i):
    base = lax.shift_left(lax.shift_right_logical(i, 4), 4)
    v = ref[pl.ds(base, 16)]
    return _splat_lane(v, lax.bitwise_and(i, 15))


def _sortable(b):
    s = lax.shift_right_arithmetic(b, 31)
    return lax.bitwise_xor(b, lax.bitwise_and(s, jnp.int32(0x7FFFFFFF)))


def _sc_body(logits_hbm, pres_hbm, freq_hbm, rep_hbm, temp_hbm, topp_hbm,
             minp_hbm, ptok_hbm, otok_hbm, topk_hbm, out_hbm,
             buf, hist, cval, cidx, evals, ptok, otok, praw, oraw,
             pres_v, freq_v, rep_v, temp_v, topp_v, minp_v, topk_v):
    nc = 2
    wid = lax.axis_index("s") * nc + lax.axis_index("c")

    pltpu.sync_copy(pres_hbm, pres_v)
    pltpu.sync_copy(freq_hbm, freq_v)
    pltpu.sync_copy(rep_hbm, rep_v)
    pltpu.sync_copy(temp_hbm, temp_v)
    pltpu.sync_copy(topp_hbm, topp_v)
    pltpu.sync_copy(minp_hbm, minp_v)
    pltpu.sync_copy(topk_hbm, topk_v)

    iota = _iota16()
    zeros_f = _full_f(0.0)
    zeros_i = _full_i(0)
    ones_i = _full_i(1)
    ninf = _full_f(_NEG_INF)

    def do_row(rr, _):
        r = wid * 2 + rr

        pltpu.sync_copy(logits_hbm.at[pl.ds(r * _BUFN, _BUFN)], buf)
        pltpu.sync_copy(ptok_hbm.at[pl.ds(r * 512, 512)], ptok)
        pltpu.sync_copy(otok_hbm.at[pl.ds(r * 128, 128)], otok)

        rep16 = _read_splat(rep_v, r)
        freq16 = _read_splat(freq_v, r)
        pres16 = _read_splat(pres_v, r)
        t_raw = _read_splat(temp_v, r)
        t16 = jnp.where(t_raw < _full_f(1e-2), _full_f(1.0), t_raw)
        topp16 = _full_f(1.0) - _read_splat(topp_v, r)
        minp16 = _read_splat(minp_v, r)
        k_clip = jnp.clip(_read_splat(topk_v, r), 1, _P - 1)[0]

        def g_p(i, _):
            praw[pl.ds(i * 16, 16)] = plsc.load_gather(buf, [ptok[pl.ds(i * 16, 16)]])
            return 0
        lax.fori_loop(0, 32, g_p, 0)

        def g_o(i, _):
            oraw[pl.ds(i * 16, 16)] = plsc.load_gather(buf, [otok[pl.ds(i * 16, 16)]])
            return 0
        lax.fori_loop(0, 8, g_o, 0)

        def w_p(i, _):
            x = praw[pl.ds(i * 16, 16)]
            z = jnp.where(x > zeros_f, x / rep16, x * rep16)
            plsc.store_scatter(buf, [ptok[pl.ds(i * 16, 16)]], z)
            return 0
        lax.fori_loop(0, 32, w_p, 0)

        ovs = [otok[pl.ds(j * 16, 16)] for j in range(8)]

        def cnt_body(rot, accs):
            perm = lax.bitwise_and(iota + rot, _full_i(15))
            rbs = [_permute(b, perm) for b in ovs]
            out = []
            for i in range(8):
                a = accs[i]
                for j in range(8):
                    a = a + jnp.where(ovs[i] == rbs[j], ones_i, zeros_i)
                out.append(a)
            return tuple(out)
        counts = lax.fori_loop(0, 16, cnt_body, tuple(zeros_i for _ in range(8)))

        for j in range(8):
            x = oraw[pl.ds(j * 16, 16)]
            z = jnp.where(x > zeros_f, x / rep16, x * rep16)
            z = z - freq16 * counts[j].astype(jnp.float32)
            z = z - pres16
            plsc.store_scatter(buf, [ovs[j]], z)

        def h_zero(i, _):
            hist[pl.ds(i * 16, 16)] = zeros_i
            return 0
        lax.fori_loop(0, _NBINS // 16, h_zero, 0, unroll=8)

        def h_body(i, _):
            x = buf[pl.ds(i * 16, 16)]
            b = lax.shift_right_logical(
                lax.bitcast_convert_type(x, jnp.int32), _BIN_SHIFT)
            plsc.addupdate_scatter(hist, [b], ones_i)
            return 0
        lax.fori_loop(0, _NV, h_body, 0, unroll=8)

        k16 = _full_i(_KSEL)

        def s_pos(v, carry):
            total, ebits = carry
            vi = (_NBINS // 32 - 1) - v
            h = hist[pl.ds(vi * 16, 16)]
            rc = lax.rev(_cumsum16(lax.rev(h, (0,))), (0,))
            tot_here = _splat_lane(rc, 0)
            cv = (total + rc) >= k16
            pc = plsc.all_reduce_population_count(cv)
            b_here = _full_i(vi * 16) + pc - ones_i
            crossing = jnp.logical_and(total < k16, (total + tot_here) >= k16)
            ebits = jnp.where(crossing, lax.shift_left(b_here, _BIN_SHIFT), ebits)
            return total + tot_here, ebits

        def s_neg(v, carry):
            total, ebits = carry
            vi = (_NBINS // 32) + v
            h = hist[pl.ds(vi * 16, 16)]
            pf = _cumsum16(h)
            tot_here = _splat_lane(pf, 15)
            cv = (total + pf) >= k16
            pc = plsc.all_reduce_population_count(cv)
            b_here = _full_i(vi * 16 + 16) - pc
            crossing = jnp.logical_and(total < k16, (total + tot_here) >= k16)
            ebits = jnp.where(
                crossing,
                lax.bitwise_or(lax.shift_left(b_here, _BIN_SHIFT),
                               _full_i((1 << _BIN_SHIFT) - 1)),
                ebits)
            return total + tot_here, ebits

        carry = lax.fori_loop(0, _NBINS // 32, s_pos, (zeros_i, zeros_i), unroll=4)
        _, ebits = lax.fori_loop(0, _NBINS // 32, s_neg, carry, unroll=4)
        thresh16 = lax.bitcast_convert_type(ebits, jnp.float32)

        def c_init(i, _):
            cval[pl.ds(i * 16, 16)] = ninf
            cidx[pl.ds(i * 16, 16)] = _full_i(_VOCAB)
            return 0
        lax.fori_loop(0, _PV + 1, c_init, 0, unroll=4)

        def c_body(i, carry):
            off, iv = carry
            x = buf[pl.ds(i * 16, 16)]
            m = x >= thresh16
            off_use = jnp.minimum(off, _P)
            plsc.store_compressed(cval.at[pl.ds(off_use, 16)], x, mask=m)
            plsc.store_compressed(cidx.at[pl.ds(off_use, 16)], iv, mask=m)
            return off + plsc.all_reduce_population_count(m)[0], iv + _full_i(16)
        lax.fori_loop(0, _NV, c_body, (jnp.int32(0), iota), unroll=4)

        def t_body(i, _):
            cval[pl.ds(i * 16, 16)] = cval[pl.ds(i * 16, 16)] / t16
            return 0
        lax.fori_loop(0, _PV, t_body, 0, unroll=4)

        def vsort_dir(v, kdesc):
            kv = cval[pl.ds(v * 16, 16)]
            vv = cidx[pl.ds(v * 16, 16)]
            sk, sv = plsc.sort_key_val(kv, vv)
            ridx = jnp.where(jnp.full((16,), kdesc, dtype=bool),
                             _full_i(15) - iota, iota)
            cval[pl.ds(v * 16, 16)] = _permute(sk, ridx)
            cidx[pl.ds(v * 16, 16)] = _permute(sv, ridx)

        def init_sort(v, _):
            vsort_dir(v, lax.bitwise_and(v, 1) == 1)
            return 0
        lax.fori_loop(0, _PV, init_sort, 0, unroll=4)

        for k_el in [32, 64, 128, 256, 512, 1024, 2048]:
            kv16 = k_el // 16
            j_el = k_el // 2
            while j_el >= 16:
                jv = j_el // 16
                lg = jv.bit_length() - 1

                def ce_body(p, _, jv=jv, lg=lg, kv16=kv16):
                    v1 = lax.bitwise_or(
                        lax.shift_left(lax.shift_right_logical(p, lg), lg + 1),
                        lax.bitwise_and(p, jv - 1))
                    v2 = v1 + jv
                    asc = lax.bitwise_and(v1, kv16) == 0
                    a = cval[pl.ds(v1 * 16, 16)]
                    b = cval[pl.ds(v2 * 16, 16)]
                    ai = cidx[pl.ds(v1 * 16, 16)]
                    bi = cidx[pl.ds(v2 * 16, 16)]
                    asc16 = jnp.full((16,), asc, dtype=bool)
                    sel = (a <= b) == asc16
                    cval[pl.ds(v1 * 16, 16)] = jnp.where(sel, a, b)
                    cval[pl.ds(v2 * 16, 16)] = jnp.where(sel, b, a)
                    cidx[pl.ds(v1 * 16, 16)] = jnp.where(sel, ai, bi)
                    cidx[pl.ds(v2 * 16, 16)] = jnp.where(sel, bi, ai)
                    return 0
                lax.fori_loop(0, _PV // 2, ce_body, 0, unroll=2)
                j_el //= 2

            def cl_body(v, _, kv16=kv16):
                vsort_dir(v, lax.bitwise_and(v, kv16) != 0)
                return 0
            lax.fori_loop(0, _PV, cl_body, 0, unroll=4)

        pos = _P - k_clip
        base = lax.shift_left(lax.shift_right_logical(pos, 4), 4)
        kth16 = _splat_lane(cval[pl.ds(base, 16)], lax.bitwise_and(pos, 15))
        m16 = _splat_lane(cval[pl.ds(_P - 16, 16)], 15)

        def pa(v, zc):
            x = cval[pl.ds(v * 16, 16)]
            x = jnp.where(x < kth16, ninf, x)
            e = jnp.exp(x - m16)
            evals[pl.ds(v * 16, 16)] = e
            return zc + _splat_lane(_cumsum16(e), 15)
        z1 = lax.fori_loop(0, _PV, pa, zeros_f, unroll=2)

        def pb(v, carry):
            sc, z2c = carry
            e = evals[pl.ds(v * 16, 16)]
            p = e / z1
            cp = _cumsum16(p)
            s = sc + cp
            e2 = jnp.where(s <= topp16, zeros_f, e)
            evals[pl.ds(v * 16, 16)] = e2
            return sc + _splat_lane(cp, 15), z2c + _splat_lane(_cumsum16(e2), 15)
        _, z2 = lax.fori_loop(0, _PV, pb, (zeros_f, zeros_f), unroll=2)

        rhs = minp16 * (_full_f(1.0) / z2)

        def pc(v, z3c):
            e2 = evals[pl.ds(v * 16, 16)]
            p2 = e2 / z2
            e3 = jnp.where(p2 < rhs, zeros_f, e2)
            evals[pl.ds(v * 16, 16)] = e3
            return z3c + _splat_lane(_cumsum16(e3), 15)
        z3 = lax.fori_loop(0, _PV, pc, zeros_f, unroll=2)

        def zb(i, _):
            buf[pl.ds(i * 16, 16)] = zeros_f
            return 0
        lax.fori_loop(0, _NV, zb, 0, unroll=8)

        def sc_out(v, _):
            plsc.store_scatter(buf, [cidx[pl.ds(v * 16, 16)]],
                               evals[pl.ds(v * 16, 16)] / z3)
            return 0
        lax.fori_loop(0, _PV, sc_out, 0, unroll=4)

        pltpu.sync_copy(buf, out_hbm.at[pl.ds(r * _BUFN, _BUFN)])
        return 0

    lax.fori_loop(0, 2, do_row, 0)


@jax.jit
def kernel(logits, presence_penalties, frequency_penalties,
           repetition_penalties, temperatures, top_p, min_p,
           prompt_tokens, output_tokens, top_k):
    mesh = plsc.VectorSubcoreMesh(core_axis_name="c", subcore_axis_name="s")
    f = pl.kernel(
        _sc_body,
        out_type=jax.ShapeDtypeStruct((_NSEQ * _BUFN,), jnp.float32),
        mesh=mesh,
        compiler_params=pltpu.CompilerParams(needs_layout_passes=False),
        scratch_types=[
            pltpu.VMEM((_BUFN,), jnp.float32),
            pltpu.VMEM((_NBINS,), jnp.int32),
            pltpu.VMEM((_P + 16,), jnp.float32),
            pltpu.VMEM((_P + 16,), jnp.int32),
            pltpu.VMEM((_P,), jnp.float32),
            pltpu.VMEM((512,), jnp.int32),
            pltpu.VMEM((128,), jnp.int32),
            pltpu.VMEM((512,), jnp.float32),
            pltpu.VMEM((128,), jnp.float32),
            pltpu.VMEM((128,), jnp.float32),
            pltpu.VMEM((128,), jnp.float32),
            pltpu.VMEM((128,), jnp.float32),
            pltpu.VMEM((128,), jnp.float32),
            pltpu.VMEM((128,), jnp.float32),
            pltpu.VMEM((128,), jnp.float32),
            pltpu.VMEM((128,), jnp.int32),
        ],
    )
    logits_p = jnp.pad(logits, ((0, 0), (0, _BUFN - _VOCAB)),
                       constant_values=_NEG_INF).reshape(-1)
    pad1 = lambda a: jnp.pad(a, (0, 128 - _NSEQ))
    out = f(logits_p, pad1(presence_penalties), pad1(frequency_penalties),
            pad1(repetition_penalties), pad1(temperatures), pad1(top_p),
            pad1(min_p), prompt_tokens.reshape(-1), output_tokens.reshape(-1),
            pad1(top_k))
    return out.reshape(_NSEQ, _BUFN)[:, :_VOCAB]
